# initial kernel scaffold (unmeasured)
import jax
import jax.numpy as jnp
from jax import lax
from jax.experimental import pallas as pl
from jax.experimental.pallas import tpu as pltpu


def kernel(
    x,
):
    def body(*refs):
        pass

    out_shape = jax.ShapeDtypeStruct(..., jnp.float32)
    return pl.pallas_call(body, out_shape=out_shape)(...)


# baseline (device time: 123610 ns/iter reference)
import jax
import jax.numpy as jnp
from jax import lax
from jax.experimental import pallas as pl
from jax.experimental.pallas import tpu as pltpu

M, N = 2048, 1024
BF16 = jnp.bfloat16


def kernel(x):
    x = x.reshape(M, N)

    def body(x_ref, out_ref, acc_ref, rb0, rb1, rb2, rb3, send_sems, recv_sems):
        my_x = lax.axis_index("x")
        my_y = lax.axis_index("y")
        my_z = lax.axis_index("z")
        z_hi = my_z // 2
        z_lo = my_z % 2

        acc_ref[...] = x_ref[...].astype(BF16)

        rs_rounds = [
            ((1 - my_x, my_y, my_z), my_x),
            ((my_x, 1 - my_y, my_z), my_y),
            ((my_x, my_y, my_z ^ 2), z_hi),
            ((my_x, my_y, my_z ^ 1), z_lo),
        ]
        recv_bufs = [rb0, rb1, rb2, rb3]

        off = jnp.int32(0)
        seg = M
        for r, (partner, b) in enumerate(rs_rounds):
            half = seg // 2
            keep_off = off + b * half
            send_off = off + (1 - b) * half
            rdma = pltpu.make_async_remote_copy(
                src_ref=acc_ref.at[pl.ds(send_off, half), :],
                dst_ref=recv_bufs[r],
                send_sem=send_sems.at[r],
                recv_sem=recv_sems.at[r],
                device_id=partner,
                device_id_type=pl.DeviceIdType.MESH,
            )
            rdma.start()
            rdma.wait()
            acc_ref[pl.ds(keep_off, half), :] = (
                acc_ref[pl.ds(keep_off, half), :] + recv_bufs[r][...]
            )
            off = keep_off
            seg = half

        for i, (partner, b) in enumerate(reversed(rs_rounds)):
            r = 4 + i
            rdma = pltpu.make_async_remote_copy(
                src_ref=acc_ref.at[pl.ds(off, seg), :],
                dst_ref=acc_ref.at[pl.ds(off, seg), :],
                send_sem=send_sems.at[r],
                recv_sem=recv_sems.at[r],
                device_id=partner,
                device_id_type=pl.DeviceIdType.MESH,
            )
            rdma.start()
            rdma.wait()
            off = off - b * seg
            seg = seg * 2

        out_ref[...] = acc_ref[...]

    return pl.pallas_call(
        body,
        out_shape=jax.ShapeDtypeStruct((M, N), BF16),
        in_specs=[pl.BlockSpec(memory_space=pltpu.VMEM)],
        out_specs=pl.BlockSpec(memory_space=pltpu.VMEM),
        scratch_shapes=[
            pltpu.VMEM((M, N), BF16),
            pltpu.VMEM((M // 2, N), BF16),
            pltpu.VMEM((M // 4, N), BF16),
            pltpu.VMEM((M // 8, N), BF16),
            pltpu.VMEM((M // 16, N), BF16),
            pltpu.SemaphoreType.DMA((8,)),
            pltpu.SemaphoreType.DMA((8,)),
        ],
    )(x)


# device time: 83431 ns/iter; 1.4816x vs baseline; 1.4816x over previous
import jax
import jax.numpy as jnp
from jax import lax
from jax.experimental import pallas as pl
from jax.experimental.pallas import tpu as pltpu

M, N = 2048, 1024
S = M // 2
BF16 = jnp.bfloat16

RB_OFFS = (0, 512, 768, 896)
RB_ROWS = 960


def kernel(x):
    x = x.reshape(M, N)

    def body(x_ref, out_ref, acc_ref, rbA, rbB, rs_send, rs_recv, ag_send, ag_recv):
        my_x = lax.axis_index("x")
        my_y = lax.axis_index("y")
        my_z = lax.axis_index("z")
        z_hi = my_z // 2
        z_lo = my_z % 2

        acc_ref[...] = x_ref[...].astype(BF16)

        axis_info = {
            "x": ((1 - my_x, my_y, my_z), my_x),
            "y": ((my_x, 1 - my_y, my_z), my_y),
            "z1": ((my_x, my_y, my_z ^ 1), z_lo),
            "z2": ((my_x, my_y, my_z ^ 2), z_hi),
        }
        chans = [
            {"rounds": [axis_info[a] for a in ("x", "y", "z1", "z2")], "rb": rbA},
            {"rounds": [axis_info[a] for a in ("z1", "z2", "x", "y")], "rb": rbB},
        ]

        offs = [jnp.int32(0), jnp.int32(S)]
        segs = [S, S]

        def start_rs(c, r):
            half = segs[c] // 2
            partner, b = chans[c]["rounds"][r]
            send_off = offs[c] + (1 - b) * half
            rdma = pltpu.make_async_remote_copy(
                src_ref=acc_ref.at[pl.ds(send_off, half), :],
                dst_ref=chans[c]["rb"].at[pl.ds(RB_OFFS[r], half), :],
                send_sem=rs_send.at[c, r],
                recv_sem=rs_recv.at[c, r],
                device_id=partner,
                device_id_type=pl.DeviceIdType.MESH,
            )
            rdma.start()
            return rdma

        infl = [start_rs(0, 0), start_rs(1, 0)]
        for r in range(4):
            for c in (0, 1):
                infl[c].wait()
                half = segs[c] // 2
                _, b = chans[c]["rounds"][r]
                keep_off = offs[c] + b * half
                acc_ref[pl.ds(keep_off, half), :] = (
                    acc_ref[pl.ds(keep_off, half), :]
                    + chans[c]["rb"][pl.ds(RB_OFFS[r], half), :]
                )
                offs[c] = keep_off
                segs[c] = half
                if r < 3:
                    infl[c] = start_rs(c, r + 1)

        ag_rounds = [list(reversed(chans[c]["rounds"])) for c in (0, 1)]

        def start_ag(c, r):
            partner, _ = ag_rounds[c][r]
            rdma = pltpu.make_async_remote_copy(
                src_ref=acc_ref.at[pl.ds(offs[c], segs[c]), :],
                dst_ref=acc_ref.at[pl.ds(offs[c], segs[c]), :],
                send_sem=ag_send.at[c, r],
                recv_sem=ag_recv.at[c, r],
                device_id=partner,
                device_id_type=pl.DeviceIdType.MESH,
            )
            rdma.start()
            return rdma

        infl = [start_ag(0, 0), start_ag(1, 0)]
        for r in range(4):
            for c in (0, 1):
                infl[c].wait()
                _, b = ag_rounds[c][r]
                offs[c] = offs[c] - b * segs[c]
                segs[c] = segs[c] * 2
                if r < 3:
                    infl[c] = start_ag(c, r + 1)

        out_ref[...] = acc_ref[...]

    return pl.pallas_call(
        body,
        out_shape=jax.ShapeDtypeStruct((M, N), BF16),
        in_specs=[pl.BlockSpec(memory_space=pltpu.VMEM)],
        out_specs=pl.BlockSpec(memory_space=pltpu.VMEM),
        scratch_shapes=[
            pltpu.VMEM((M, N), BF16),
            pltpu.VMEM((RB_ROWS, N), BF16),
            pltpu.VMEM((RB_ROWS, N), BF16),
            pltpu.SemaphoreType.DMA((2, 4)),
            pltpu.SemaphoreType.DMA((2, 4)),
            pltpu.SemaphoreType.DMA((2, 4)),
            pltpu.SemaphoreType.DMA((2, 4)),
        ],
    )(x)


# device time: 78733 ns/iter; 1.5700x vs baseline; 1.0597x over previous
import jax
import jax.numpy as jnp
from jax import lax
from jax.experimental import pallas as pl
from jax.experimental.pallas import tpu as pltpu

M, N = 2048, 1024
BF16 = jnp.bfloat16

SA, SB = 1280, 768
QA = SA // 4
HA1, HA2 = QA // 2, QA // 4
HB1, HB2 = SB // 2, SB // 4
QB = HB2 // 4


def kernel(x):
    x = x.reshape(M, N)

    def body(x_ref, out_ref, acc_ref, rbxyA, rbxyB, rbzA, rbzB,
             xy_send, xy_recv, z_send, z_recv):
        my_x = lax.axis_index("x")
        my_y = lax.axis_index("y")
        my_z = lax.axis_index("z")
        z_lo = my_z % 2
        z_hi = my_z // 2

        acc_ref[...] = x_ref[...].astype(BF16)

        xy_peers = [
            (1 - my_x, my_y, my_z),
            (my_x, 1 - my_y, my_z),
            (1 - my_x, 1 - my_y, my_z),
        ]
        q_me = my_x * 2 + my_y
        q_peer = [
            (1 - my_x) * 2 + my_y,
            my_x * 2 + (1 - my_y),
            (1 - my_x) * 2 + (1 - my_y),
        ]
        z1_peer = (my_x, my_y, my_z ^ 1)
        z2_peer = (my_x, my_y, my_z ^ 2)

        def rcopy(src, dst, ssem, rsem, dev):
            r = pltpu.make_async_remote_copy(
                src_ref=src, dst_ref=dst, send_sem=ssem, recv_sem=rsem,
                device_id=dev, device_id_type=pl.DeviceIdType.MESH,
            )
            r.start()
            return r

        def xy_scatter(ch, phase, base, q, rb):
            return [
                rcopy(
                    acc_ref.at[pl.ds(base + q_peer[s] * q, q), :],
                    rb.at[pl.ds(s * q, q), :],
                    xy_send.at[ch, phase, s], xy_recv.at[ch, phase, s],
                    xy_peers[s],
                )
                for s in range(3)
            ]

        def xy_bcast(ch, phase, off, q):
            return [
                rcopy(
                    acc_ref.at[pl.ds(off, q), :],
                    acc_ref.at[pl.ds(off, q), :],
                    xy_send.at[ch, phase, s], xy_recv.at[ch, phase, s],
                    xy_peers[s],
                )
                for s in range(3)
            ]

        a_infl = xy_scatter(0, 0, 0, QA, rbxyA)
        b_infl = rcopy(
            acc_ref.at[pl.ds(SA + (1 - z_lo) * HB1, HB1), :],
            rbzB.at[pl.ds(0, HB1), :],
            z_send.at[1, 0, 0], z_recv.at[1, 0, 0], z1_peer,
        )

        b_infl.wait()
        offB = SA + z_lo * HB1
        acc_ref[pl.ds(offB, HB1), :] = (
            acc_ref[pl.ds(offB, HB1), :] + rbzB[pl.ds(0, HB1), :]
        )
        b_infl = rcopy(
            acc_ref.at[pl.ds(offB + (1 - z_hi) * HB2, HB2), :],
            rbzB.at[pl.ds(HB1, HB2), :],
            z_send.at[1, 0, 1], z_recv.at[1, 0, 1], z2_peer,
        )

        for r in a_infl:
            r.wait()
        offA = q_me * QA
        acc_ref[pl.ds(offA, QA), :] = (
            acc_ref[pl.ds(offA, QA), :]
            + rbxyA[pl.ds(0, QA), :]
            + rbxyA[pl.ds(QA, QA), :]
            + rbxyA[pl.ds(2 * QA, QA), :]
        )
        a_infl = rcopy(
            acc_ref.at[pl.ds(offA + (1 - z_lo) * HA1, HA1), :],
            rbzA.at[pl.ds(0, HA1), :],
            z_send.at[0, 0, 0], z_recv.at[0, 0, 0], z1_peer,
        )

        b_infl.wait()
        offB = offB + z_hi * HB2
        acc_ref[pl.ds(offB, HB2), :] = (
            acc_ref[pl.ds(offB, HB2), :] + rbzB[pl.ds(HB1, HB2), :]
        )
        b_xy = xy_scatter(1, 0, offB, QB, rbxyB)

        a_infl.wait()
        offA = offA + z_lo * HA1
        acc_ref[pl.ds(offA, HA1), :] = (
            acc_ref[pl.ds(offA, HA1), :] + rbzA[pl.ds(0, HA1), :]
        )
        a_infl = rcopy(
            acc_ref.at[pl.ds(offA + (1 - z_hi) * HA2, HA2), :],
            rbzA.at[pl.ds(HA1, HA2), :],
            z_send.at[0, 0, 1], z_recv.at[0, 0, 1], z2_peer,
        )

        for r in b_xy:
            r.wait()
        offB2 = offB + q_me * QB
        acc_ref[pl.ds(offB2, QB), :] = (
            acc_ref[pl.ds(offB2, QB), :]
            + rbxyB[pl.ds(0, QB), :]
            + rbxyB[pl.ds(QB, QB), :]
            + rbxyB[pl.ds(2 * QB, QB), :]
        )

        b_xy = xy_bcast(1, 1, offB2, QB)

        a_infl.wait()
        offA2 = offA + z_hi * HA2
        acc_ref[pl.ds(offA2, HA2), :] = (
            acc_ref[pl.ds(offA2, HA2), :] + rbzA[pl.ds(HA1, HA2), :]
        )
        a_infl = rcopy(
            acc_ref.at[pl.ds(offA2, HA2), :],
            acc_ref.at[pl.ds(offA2, HA2), :],
            z_send.at[0, 1, 1], z_recv.at[0, 1, 1], z2_peer,
        )

        for r in b_xy:
            r.wait()
        b_infl = rcopy(
            acc_ref.at[pl.ds(offB, HB2), :],
            acc_ref.at[pl.ds(offB, HB2), :],
            z_send.at[1, 1, 1], z_recv.at[1, 1, 1], z2_peer,
        )

        a_infl.wait()
        a_infl = rcopy(
            acc_ref.at[pl.ds(offA, HA1), :],
            acc_ref.at[pl.ds(offA, HA1), :],
            z_send.at[0, 1, 0], z_recv.at[0, 1, 0], z1_peer,
        )

        b_infl.wait()
        offB = offB - z_hi * HB2
        b_infl = rcopy(
            acc_ref.at[pl.ds(offB, HB1), :],
            acc_ref.at[pl.ds(offB, HB1), :],
            z_send.at[1, 1, 0], z_recv.at[1, 1, 0], z1_peer,
        )

        a_infl.wait()
        a_xy = xy_bcast(0, 1, q_me * QA, QA)

        b_infl.wait()
        for r in a_xy:
            r.wait()

        out_ref[...] = acc_ref[...]

    return pl.pallas_call(
        body,
        out_shape=jax.ShapeDtypeStruct((M, N), BF16),
        in_specs=[pl.BlockSpec(memory_space=pltpu.VMEM)],
        out_specs=pl.BlockSpec(memory_space=pltpu.VMEM),
        scratch_shapes=[
            pltpu.VMEM((M, N), BF16),
            pltpu.VMEM((3 * QA, N), BF16),
            pltpu.VMEM((3 * QB, N), BF16),
            pltpu.VMEM((HA1 + HA2, N), BF16),
            pltpu.VMEM((HB1 + HB2, N), BF16),
            pltpu.SemaphoreType.DMA((2, 2, 3)),
            pltpu.SemaphoreType.DMA((2, 2, 3)),
            pltpu.SemaphoreType.DMA((2, 2, 2)),
            pltpu.SemaphoreType.DMA((2, 2, 2)),
        ],
    )(x)


# device time: 60747 ns/iter; 2.0348x vs baseline; 1.2961x over previous
import jax
import jax.numpy as jnp
from jax import lax
from jax.experimental import pallas as pl
from jax.experimental.pallas import tpu as pltpu

M, N = 2048, 1024
BF16 = jnp.bfloat16

SA, SB = 1280, 768
QA = SA // 4
HA1, HA2 = QA // 2, QA // 4
HB1, HB2 = SB // 2, SB // 4
QB = HB2 // 4


def kernel(x):
    x = x.reshape(M, N)

    def body(x_ref, out_ref, rbxyA, rbxyB, rbzA, rbzB,
             xy_send, xy_recv, z_send, z_recv):
        my_x = lax.axis_index("x")
        my_y = lax.axis_index("y")
        my_z = lax.axis_index("z")
        z_lo = my_z % 2
        z_hi = my_z // 2

        acc = out_ref
        acc[...] = x_ref[...].astype(BF16)

        xy_peers = [
            (1 - my_x, my_y, my_z),
            (my_x, 1 - my_y, my_z),
            (1 - my_x, 1 - my_y, my_z),
        ]
        q_me = my_x * 2 + my_y
        q_peer = [
            (1 - my_x) * 2 + my_y,
            my_x * 2 + (1 - my_y),
            (1 - my_x) * 2 + (1 - my_y),
        ]
        z1_peer = (my_x, my_y, my_z ^ 1)
        z2_peer = (my_x, my_y, my_z ^ 2)

        def rcopy(src, dst, ssem, rsem, dev):
            r = pltpu.make_async_remote_copy(
                src_ref=src, dst_ref=dst, send_sem=ssem, recv_sem=rsem,
                device_id=dev, device_id_type=pl.DeviceIdType.MESH,
            )
            r.start()
            return r

        subA = [(1 - z_lo) * HA1, z_lo * HA1]
        a_xy = [
            [
                rcopy(
                    acc.at[pl.ds(q_peer[s] * QA + subA[sub], HA1), :],
                    rbxyA.at[pl.ds(s * QA + subA[sub], HA1), :],
                    xy_send.at[0, 0, sub, s], xy_recv.at[0, 0, sub, s],
                    xy_peers[s],
                )
                for s in range(3)
            ]
            for sub in (0, 1)
        ]

        keepB_pk = SA + (1 - z_lo) * HB1
        subB = [(1 - z_hi) * HB2, z_hi * HB2]
        b_z1 = [
            rcopy(
                acc.at[pl.ds(keepB_pk + subB[sub], HB2), :],
                rbzB.at[pl.ds(subB[sub], HB2), :],
                z_send.at[1, 0, 0, sub], z_recv.at[1, 0, 0, sub], z1_peer,
            )
            for sub in (0, 1)
        ]
        offB = SA + z_lo * HB1

        b_z1[0].wait()
        acc[pl.ds(offB + subB[0], HB2), :] = (
            acc[pl.ds(offB + subB[0], HB2), :] + rbzB[pl.ds(subB[0], HB2), :]
        )
        b_z2 = rcopy(
            acc.at[pl.ds(offB + subB[0], HB2), :],
            rbzB.at[pl.ds(HB1, HB2), :],
            z_send.at[1, 0, 1, 0], z_recv.at[1, 0, 1, 0], z2_peer,
        )

        for r in a_xy[0]:
            r.wait()
        offA_q = q_me * QA
        s0 = offA_q + subA[0]
        acc[pl.ds(s0, HA1), :] = (
            acc[pl.ds(s0, HA1), :]
            + rbxyA[pl.ds(0 * QA + subA[0], HA1), :]
            + rbxyA[pl.ds(1 * QA + subA[0], HA1), :]
            + rbxyA[pl.ds(2 * QA + subA[0], HA1), :]
        )
        a_z1 = rcopy(
            acc.at[pl.ds(s0, HA1), :],
            rbzA.at[pl.ds(0, HA1), :],
            z_send.at[0, 0, 0, 0], z_recv.at[0, 0, 0, 0], z1_peer,
        )

        b_z1[1].wait()
        acc[pl.ds(offB + subB[1], HB2), :] = (
            acc[pl.ds(offB + subB[1], HB2), :] + rbzB[pl.ds(subB[1], HB2), :]
        )

        for r in a_xy[1]:
            r.wait()
        offA = offA_q + z_lo * HA1
        acc[pl.ds(offA, HA1), :] = (
            acc[pl.ds(offA, HA1), :]
            + rbxyA[pl.ds(0 * QA + subA[1], HA1), :]
            + rbxyA[pl.ds(1 * QA + subA[1], HA1), :]
            + rbxyA[pl.ds(2 * QA + subA[1], HA1), :]
        )
        a_z1.wait()
        acc[pl.ds(offA, HA1), :] = (
            acc[pl.ds(offA, HA1), :] + rbzA[pl.ds(0, HA1), :]
        )
        a_z2 = rcopy(
            acc.at[pl.ds(offA + (1 - z_hi) * HA2, HA2), :],
            rbzA.at[pl.ds(HA1, HA2), :],
            z_send.at[0, 0, 1, 0], z_recv.at[0, 0, 1, 0], z2_peer,
        )

        b_z2.wait()
        offB = offB + z_hi * HB2
        acc[pl.ds(offB, HB2), :] = (
            acc[pl.ds(offB, HB2), :] + rbzB[pl.ds(HB1, HB2), :]
        )
        b_xy = [
            rcopy(
                acc.at[pl.ds(offB + q_peer[s] * QB, QB), :],
                rbxyB.at[pl.ds(s * QB, QB), :],
                xy_send.at[1, 0, 0, s], xy_recv.at[1, 0, 0, s],
                xy_peers[s],
            )
            for s in range(3)
        ]

        a_z2.wait()
        offA2 = offA + z_hi * HA2
        acc[pl.ds(offA2, HA2), :] = (
            acc[pl.ds(offA2, HA2), :] + rbzA[pl.ds(HA1, HA2), :]
        )

        a_z2 = rcopy(
            acc.at[pl.ds(offA2, HA2), :],
            acc.at[pl.ds(offA2, HA2), :],
            z_send.at[0, 1, 1, 0], z_recv.at[0, 1, 1, 0], z2_peer,
        )

        for r in b_xy:
            r.wait()
        offB2 = offB + q_me * QB
        acc[pl.ds(offB2, QB), :] = (
            acc[pl.ds(offB2, QB), :]
            + rbxyB[pl.ds(0, QB), :]
            + rbxyB[pl.ds(QB, QB), :]
            + rbxyB[pl.ds(2 * QB, QB), :]
        )
        b_xy = [
            rcopy(
                acc.at[pl.ds(offB2, QB), :],
                acc.at[pl.ds(offB2, QB), :],
                xy_send.at[1, 1, 0, s], xy_recv.at[1, 1, 0, s],
                xy_peers[s],
            )
            for s in range(3)
        ]

        a_z1_wait = a_z2
        a_z1_wait.wait()
        a_z1 = rcopy(
            acc.at[pl.ds(offA, HA1), :],
            acc.at[pl.ds(offA, HA1), :],
            z_send.at[0, 1, 0, 0], z_recv.at[0, 1, 0, 0], z1_peer,
        )
        a_xy1 = [
            rcopy(
                acc.at[pl.ds(offA, HA1), :],
                acc.at[pl.ds(offA, HA1), :],
                xy_send.at[0, 1, 0, s], xy_recv.at[0, 1, 0, s],
                xy_peers[s],
            )
            for s in range(3)
        ]

        for r in b_xy:
            r.wait()
        offB_half = SA + z_lo * HB1
        b_z2 = rcopy(
            acc.at[pl.ds(offB, HB2), :],
            acc.at[pl.ds(offB, HB2), :],
            z_send.at[1, 1, 1, 0], z_recv.at[1, 1, 1, 0], z2_peer,
        )
        b_z1a = rcopy(
            acc.at[pl.ds(offB, HB2), :],
            acc.at[pl.ds(offB, HB2), :],
            z_send.at[1, 1, 0, 0], z_recv.at[1, 1, 0, 0], z1_peer,
        )

        a_z1.wait()
        offA_o = offA_q + (1 - z_lo) * HA1
        a_xy2 = [
            rcopy(
                acc.at[pl.ds(offA_o, HA1), :],
                acc.at[pl.ds(offA_o, HA1), :],
                xy_send.at[0, 1, 1, s], xy_recv.at[0, 1, 1, s],
                xy_peers[s],
            )
            for s in range(3)
        ]

        b_z2.wait()
        offB_z2o = offB_half + (1 - z_hi) * HB2
        b_z1b = rcopy(
            acc.at[pl.ds(offB_z2o, HB2), :],
            acc.at[pl.ds(offB_z2o, HB2), :],
            z_send.at[1, 1, 0, 1], z_recv.at[1, 1, 0, 1], z1_peer,
        )

        b_z1a.wait()
        b_z1b.wait()
        for r in a_xy1:
            r.wait()
        for r in a_xy2:
            r.wait()

    return pl.pallas_call(
        body,
        out_shape=jax.ShapeDtypeStruct((M, N), BF16),
        in_specs=[pl.BlockSpec(memory_space=pltpu.VMEM)],
        out_specs=pl.BlockSpec(memory_space=pltpu.VMEM),
        scratch_shapes=[
            pltpu.VMEM((3 * QA, N), BF16),
            pltpu.VMEM((3 * QB, N), BF16),
            pltpu.VMEM((HA1 + HA2, N), BF16),
            pltpu.VMEM((HB1 + HB2, N), BF16),
            pltpu.SemaphoreType.DMA((2, 2, 2, 3)),
            pltpu.SemaphoreType.DMA((2, 2, 2, 3)),
            pltpu.SemaphoreType.DMA((2, 2, 2, 2)),
            pltpu.SemaphoreType.DMA((2, 2, 2, 2)),
        ],
    )(x)
